# Initial kernel scaffold; baseline (speedup 1.0000x reference)
import jax
import jax.numpy as jnp
from jax.experimental import pallas as pl


def kernel(x, edge_index, W1, b1, W2, att_src, att_dst, b2, Wfc, bfc):
    raise NotImplementedError("write your pallas kernel here")


# speedup vs baseline: 49.6143x; 49.6143x over previous
import functools

import jax
import jax.numpy as jnp
from jax import lax
from jax.experimental import pallas as pl
from jax.experimental.pallas import tpu as pltpu
from jax.experimental.pallas import tpu_sc as plsc

_N, _E, _D, _H = 10000, 320000, 128, 64
_H2 = _H // 2

_NPAD = 10240
_CHUNK = 128
_NTILES = 32
_CPT = 80
_EPAD = _NTILES * _CPT * _CHUNK
_NCHUNKS = _EPAD // _CHUNK
_ROWS_PT = _NPAD // 16

_MESH = plsc.VectorSubcoreMesh(core_axis_name="c", subcore_axis_name="s")
_SC_PARAMS = pltpu.CompilerParams(
    needs_layout_passes=False, use_tc_tiling_on_sc=False)


def _zero16(ref, words):
    z = jnp.zeros((16,), jnp.float32)

    def body(i, _):
        ref[pl.ds(i * 16, 16)] = z
        return _

    lax.fori_loop(0, words // 16, body, None)


@functools.partial(
    pl.kernel,
    mesh=_MESH,
    compiler_params=_SC_PARAMS,
    out_type=jax.ShapeDtypeStruct((2, _NPAD), jnp.float32),
    scratch_types=[
        pltpu.VMEM((_CPT, _CHUNK), jnp.int32),
        pltpu.VMEM((_CHUNK,), jnp.float32),
        pltpu.VMEM((_ROWS_PT,), jnp.float32),
        pltpu.VMEM_SHARED((_NPAD,), jnp.float32),
    ],
)
def _sc_degree(col_hbm, out_hbm, col_v, ones_v, buf_v, acc_sh):
    cid = lax.axis_index("c")
    sid = lax.axis_index("s")
    wid = cid * 16 + sid

    pltpu.sync_copy(col_hbm.at[pl.ds(wid * _CPT, _CPT)], col_v)
    one = jnp.ones((16,), jnp.float32)
    for k in range(_CHUNK // 16):
        ones_v[pl.ds(k * 16, 16)] = one
    _zero16(buf_v, _ROWS_PT)
    pltpu.sync_copy(buf_v, acc_sh.at[pl.ds(sid * _ROWS_PT, _ROWS_PT)])
    plsc.subcore_barrier()

    def body(j, _):
        pltpu.sync_copy(ones_v, acc_sh.at[col_v.at[j]], add=True)
        return _

    lax.fori_loop(0, _CPT, body, None)
    plsc.subcore_barrier()

    pltpu.sync_copy(acc_sh.at[pl.ds(sid * _ROWS_PT, _ROWS_PT)], buf_v)
    pltpu.sync_copy(buf_v, out_hbm.at[cid, pl.ds(sid * _ROWS_PT, _ROWS_PT)])


@functools.partial(
    pl.kernel,
    mesh=_MESH,
    compiler_params=_SC_PARAMS,
    out_type=jax.ShapeDtypeStruct((2, _NPAD, _H), jnp.float32),
    scratch_types=[
        pltpu.VMEM((_CPT, _CHUNK), jnp.int32),
        pltpu.VMEM((_CPT, _CHUNK), jnp.int32),
        pltpu.VMEM((_CHUNK, _H), jnp.float32),
        pltpu.VMEM((40, _H), jnp.float32),
        pltpu.VMEM_SHARED((_NPAD, _H), jnp.float32),
        pltpu.SemaphoreType.DMA,
    ],
)
def _sc_gcn(row_hbm, col_hbm, g_hbm, out_hbm, row_v, col_v, rowbuf,
            zbuf, acc_sh, sem):
    cid = lax.axis_index("c")
    sid = lax.axis_index("s")
    wid = cid * 16 + sid

    pltpu.sync_copy(row_hbm.at[pl.ds(wid * _CPT, _CPT)], row_v)
    pltpu.sync_copy(col_hbm.at[pl.ds(wid * _CPT, _CPT)], col_v)
    z = jnp.zeros((16,), jnp.float32)
    for r in range(40):
        for k in range(_H // 16):
            zbuf[r, pl.ds(k * 16, 16)] = z
    for i in range(16):
        pltpu.sync_copy(zbuf, acc_sh.at[pl.ds(sid * _ROWS_PT + i * 40, 40)])
    plsc.subcore_barrier()

    def body(j, _):
        pltpu.async_copy(g_hbm.at[row_v.at[j]], rowbuf, sem).wait()
        pltpu.sync_copy(rowbuf, acc_sh.at[col_v.at[j]], add=True)
        return _

    lax.fori_loop(0, _CPT, body, None)
    plsc.subcore_barrier()

    for i in range(16):
        pltpu.sync_copy(acc_sh.at[pl.ds(sid * _ROWS_PT + i * 40, 40)], zbuf)
        pltpu.sync_copy(zbuf, out_hbm.at[cid, pl.ds(sid * _ROWS_PT + i * 40, 40)])


@functools.partial(
    pl.kernel,
    mesh=_MESH,
    compiler_params=_SC_PARAMS,
    out_type=(
        jax.ShapeDtypeStruct((2, _NPAD, _H2), jnp.float32),
        jax.ShapeDtypeStruct((2, _NPAD), jnp.float32),
    ),
    scratch_types=[
        pltpu.VMEM((_CPT, _CHUNK), jnp.int32),
        pltpu.VMEM((_CPT, _CHUNK), jnp.int32),
        pltpu.VMEM((_NPAD,), jnp.float32),
        pltpu.VMEM((_NPAD,), jnp.float32),
        pltpu.VMEM((16,), jnp.float32),
        pltpu.VMEM((_CHUNK, _H2), jnp.float32),
        pltpu.VMEM((_CHUNK,), jnp.float32),
        pltpu.VMEM((40, _H2), jnp.float32),
        pltpu.VMEM((_ROWS_PT,), jnp.float32),
        pltpu.VMEM_SHARED((_NPAD, _H2), jnp.float32),
        pltpu.VMEM_SHARED((_NPAD,), jnp.float32),
        pltpu.SemaphoreType.DMA,
    ],
)
def _sc_gat(row_hbm, col_hbm, h2_hbm, asrc_hbm, adst_hbm, m_hbm,
            msg_hbm, den_hbm,
            row_v, col_v, asrc_v, adst_v, m_v, rowbuf, exbuf, zbuf, zdbuf,
            accm_sh, accd_sh, sem):
    cid = lax.axis_index("c")
    sid = lax.axis_index("s")
    wid = cid * 16 + sid

    pltpu.sync_copy(row_hbm.at[pl.ds(wid * _CPT, _CPT)], row_v)
    pltpu.sync_copy(col_hbm.at[pl.ds(wid * _CPT, _CPT)], col_v)
    pltpu.sync_copy(asrc_hbm, asrc_v)
    pltpu.sync_copy(adst_hbm, adst_v)
    pltpu.sync_copy(m_hbm, m_v)

    z = jnp.zeros((16,), jnp.float32)
    for r in range(40):
        for k in range(_H2 // 16):
            zbuf[r, pl.ds(k * 16, 16)] = z
    _zero16(zdbuf, _ROWS_PT)
    for i in range(16):
        pltpu.sync_copy(zbuf, acc_m_slice(accm_sh, sid, i))
    pltpu.sync_copy(zdbuf, accd_sh.at[pl.ds(sid * _ROWS_PT, _ROWS_PT)])
    plsc.subcore_barrier()

    m = m_v[...]

    def body(j, _):
        pltpu.async_copy(h2_hbm.at[row_v.at[j]], rowbuf, sem).wait()
        for k in range(_CHUNK // 16):
            r16 = row_v[j, pl.ds(k * 16, 16)]
            c16 = col_v[j, pl.ds(k * 16, 16)]
            es = plsc.load_gather(asrc_v, [r16])
            ed = plsc.load_gather(adst_v, [c16])
            s = es + ed
            e = jnp.maximum(s, 0.2 * s)
            ex = jnp.exp(e - m)
            exbuf[pl.ds(k * 16, 16)] = ex
            for l in range(16):
                sp = ex.at[jnp.full((16,), l, jnp.int32)].get(
                    mode="promise_in_bounds")
                q = k * 16 + l
                rowbuf[q, pl.ds(0, 16)] = rowbuf[q, pl.ds(0, 16)] * sp
                rowbuf[q, pl.ds(16, 16)] = rowbuf[q, pl.ds(16, 16)] * sp
        pltpu.sync_copy(rowbuf, accm_sh.at[col_v.at[j]], add=True)
        pltpu.sync_copy(exbuf, accd_sh.at[col_v.at[j]], add=True)
        return _

    lax.fori_loop(0, _CPT, body, None)
    plsc.subcore_barrier()

    for i in range(16):
        pltpu.sync_copy(acc_m_slice(accm_sh, sid, i), zbuf)
        pltpu.sync_copy(zbuf, msg_hbm.at[cid, pl.ds(sid * _ROWS_PT + i * 40, 40)])
    pltpu.sync_copy(accd_sh.at[pl.ds(sid * _ROWS_PT, _ROWS_PT)], zdbuf)
    pltpu.sync_copy(zdbuf, den_hbm.at[cid, pl.ds(sid * _ROWS_PT, _ROWS_PT)])


def acc_m_slice(acc, sid, i):
    return acc.at[pl.ds(sid * _ROWS_PT + i * 40, 40)]


def _tc1_body(x_ref, w1_ref, degp_ref, g_ref, h_ref, dinv_ref):
    h = jnp.dot(x_ref[...], w1_ref[...], preferred_element_type=jnp.float32)
    deg = degp_ref[0] + degp_ref[1] + 1.0
    dinv = lax.rsqrt(deg)
    dinv_ref[...] = dinv
    h_ref[...] = h
    g_ref[0:_N, :] = h * dinv[0:_N]
    g_ref[_N:_NPAD, :] = jnp.zeros((_NPAD - _N, _H), jnp.float32)


def _tc2_body(h_ref, dinv_ref, msgp_ref, w2_ref, atts_ref, attd_ref, b1_ref,
              h2_ref, asrc_ref, adst_ref, m_ref):
    h = h_ref[...]
    dinv = dinv_ref[0:_N]
    msg = msgp_ref[0, 0:_N, :] + msgp_ref[1, 0:_N, :]
    h1 = jnp.maximum(dinv * (msg + dinv * h) + b1_ref[...], 0.0)
    h2 = jnp.dot(h1, w2_ref[...], preferred_element_type=jnp.float32)
    a_s = jnp.sum(h2 * atts_ref[...], axis=1, keepdims=True)
    a_d = jnp.sum(h2 * attd_ref[...], axis=1, keepdims=True)
    h2_ref[0:_N, :] = h2
    h2_ref[_N:_NPAD, :] = jnp.zeros((_NPAD - _N, _H2), jnp.float32)
    zcol = jnp.zeros((_NPAD - _N, 1), jnp.float32)
    asrc_ref[0:_N] = a_s
    asrc_ref[_N:_NPAD] = zcol
    adst_ref[0:_N] = a_d
    adst_ref[_N:_NPAD] = zcol
    smax = jnp.max(a_s) + jnp.max(a_d)
    mm = jnp.maximum(smax, 0.2 * smax)
    m_ref[...] = jnp.full((1, 16), mm, jnp.float32)


def _tc3_body(h2_ref, asrc_ref, adst_ref, m_ref, msgp_ref, denp_ref,
              b2_ref, wfc_ref, bfc_ref, out_ref):
    m = m_ref[0, 0]
    s = asrc_ref[0:_N] + adst_ref[0:_N]
    e_self = jnp.maximum(s, 0.2 * s)
    exs = jnp.exp(e_self - m)
    h2 = h2_ref[0:_N, :]
    num = msgp_ref[0, 0:_N, :] + msgp_ref[1, 0:_N, :] + exs * h2
    den = denp_ref[0, 0:_N] + denp_ref[1, 0:_N] + exs + 1e-16
    h3 = jnp.maximum(num / den + b2_ref[...], 0.0)
    out_ref[...] = (
        jnp.dot(h3, wfc_ref[...], preferred_element_type=jnp.float32)
        + bfc_ref[...]
    )


def _tc_call(body, out_shape, *args):
    return pl.pallas_call(body, out_shape=out_shape)(*args)


def kernel(x, edge_index, W1, b1, W2, att_src, att_dst, b2, Wfc, bfc):
    row = edge_index[0].astype(jnp.int32)
    col = edge_index[1].astype(jnp.int32)
    pad = _EPAD - _E
    ar = jnp.arange(pad, dtype=jnp.int32) % 128
    row_p = jnp.concatenate([row, ar]).reshape(_NCHUNKS, _CHUNK)
    col_p = jnp.concatenate([col, _N + ar]).reshape(_NCHUNKS, _CHUNK)

    degp = _sc_degree(col_p)

    g, h, dinv = _tc_call(
        _tc1_body,
        (
            jax.ShapeDtypeStruct((_NPAD, _H), jnp.float32),
            jax.ShapeDtypeStruct((_N, _H), jnp.float32),
            jax.ShapeDtypeStruct((_NPAD, 1), jnp.float32),
        ),
        x, W1, degp.reshape(2, _NPAD, 1),
    )

    msgp = _sc_gcn(row_p, col_p, g)

    h2, asrc, adst, mvec = _tc_call(
        _tc2_body,
        (
            jax.ShapeDtypeStruct((_NPAD, _H2), jnp.float32),
            jax.ShapeDtypeStruct((_NPAD, 1), jnp.float32),
            jax.ShapeDtypeStruct((_NPAD, 1), jnp.float32),
            jax.ShapeDtypeStruct((1, 16), jnp.float32),
        ),
        h, dinv, msgp, W2, att_src.reshape(1, _H2), att_dst.reshape(1, _H2),
        b1.reshape(1, _H),
    )

    msg2p, denp = _sc_gat(
        row_p, col_p, h2, asrc.reshape(_NPAD), adst.reshape(_NPAD),
        mvec.reshape(16),
    )

    out = _tc_call(
        _tc3_body,
        jax.ShapeDtypeStruct((_N, 2), jnp.float32),
        h2, asrc, adst, mvec, msg2p, denp.reshape(2, _NPAD, 1),
        b2.reshape(1, _H2), Wfc, bfc.reshape(1, 2),
    )
    return out


# speedup vs baseline: 60.7725x; 1.2249x over previous
import functools

import jax
import jax.numpy as jnp
from jax import lax
from jax.experimental import pallas as pl
from jax.experimental.pallas import tpu as pltpu
from jax.experimental.pallas import tpu_sc as plsc

_N, _E, _D, _H = 10000, 320000, 128, 64
_H2 = _H // 2

_NPAD = 10240
_CHUNK = 128
_NTILES = 32
_CPT = 80
_EPAD = _NTILES * _CPT * _CHUNK
_NCHUNKS = _EPAD // _CHUNK
_ROWS_PT = _NPAD // 16

_MESH = plsc.VectorSubcoreMesh(core_axis_name="c", subcore_axis_name="s")
_SC_PARAMS = pltpu.CompilerParams(
    needs_layout_passes=False, use_tc_tiling_on_sc=False)


def _zero16(ref, words):
    z = jnp.zeros((16,), jnp.float32)

    def body(i, _):
        ref[pl.ds(i * 16, 16)] = z
        return _

    lax.fori_loop(0, words // 16, body, None)


@functools.partial(
    pl.kernel,
    mesh=_MESH,
    compiler_params=_SC_PARAMS,
    out_type=jax.ShapeDtypeStruct((2, _NPAD), jnp.float32),
    scratch_types=[
        pltpu.VMEM((_CPT, _CHUNK), jnp.int32),
        pltpu.VMEM((_CHUNK,), jnp.float32),
        pltpu.VMEM((_ROWS_PT,), jnp.float32),
        pltpu.VMEM_SHARED((_NPAD,), jnp.float32),
    ],
)
def _sc_degree(col_hbm, out_hbm, col_v, ones_v, buf_v, acc_sh):
    cid = lax.axis_index("c")
    sid = lax.axis_index("s")
    wid = cid * 16 + sid

    pltpu.sync_copy(col_hbm.at[pl.ds(wid * _CPT, _CPT)], col_v)
    one = jnp.ones((16,), jnp.float32)
    for k in range(_CHUNK // 16):
        ones_v[pl.ds(k * 16, 16)] = one
    _zero16(buf_v, _ROWS_PT)
    pltpu.sync_copy(buf_v, acc_sh.at[pl.ds(sid * _ROWS_PT, _ROWS_PT)])
    plsc.subcore_barrier()

    def body(j, _):
        pltpu.sync_copy(ones_v, acc_sh.at[col_v.at[j]], add=True)
        return _

    lax.fori_loop(0, _CPT, body, None)
    plsc.subcore_barrier()

    pltpu.sync_copy(acc_sh.at[pl.ds(sid * _ROWS_PT, _ROWS_PT)], buf_v)
    pltpu.sync_copy(buf_v, out_hbm.at[cid, pl.ds(sid * _ROWS_PT, _ROWS_PT)])


@functools.partial(
    pl.kernel,
    mesh=_MESH,
    compiler_params=_SC_PARAMS,
    out_type=jax.ShapeDtypeStruct((2, _NPAD, _H), jnp.float32),
    scratch_types=[
        pltpu.VMEM((_CPT, _CHUNK), jnp.int32),
        pltpu.VMEM((_CPT, _CHUNK), jnp.int32),
        pltpu.VMEM((_CHUNK, _H), jnp.float32),
        pltpu.VMEM((_CHUNK, _H), jnp.float32),
        pltpu.VMEM((40, _H), jnp.float32),
        pltpu.VMEM_SHARED((_NPAD, _H), jnp.float32),
        pltpu.SemaphoreType.DMA,
        pltpu.SemaphoreType.DMA,
        pltpu.SemaphoreType.DMA,
        pltpu.SemaphoreType.DMA,
    ],
)
def _sc_gcn(row_hbm, col_hbm, g_hbm, out_hbm, row_v, col_v, bufa, bufb,
            zbuf, acc_sh, sga, sgb, ssa, ssb):
    cid = lax.axis_index("c")
    sid = lax.axis_index("s")
    wid = cid * 16 + sid

    pltpu.sync_copy(row_hbm.at[pl.ds(wid * _CPT, _CPT)], row_v)
    pltpu.sync_copy(col_hbm.at[pl.ds(wid * _CPT, _CPT)], col_v)
    z = jnp.zeros((16,), jnp.float32)
    for r in range(40):
        for k in range(_H // 16):
            zbuf[r, pl.ds(k * 16, 16)] = z
    for i in range(16):
        pltpu.sync_copy(zbuf, acc_sh.at[pl.ds(sid * _ROWS_PT + i * 40, 40)])
    plsc.subcore_barrier()

    def body(t, _):
        a = 2 * t
        b = a + 1
        ga = pltpu.async_copy(g_hbm.at[row_v.at[a]], bufa, sga)
        gb = pltpu.async_copy(g_hbm.at[row_v.at[b]], bufb, sgb)
        ga.wait()
        sa = pltpu.async_copy(bufa, acc_sh.at[col_v.at[a]], ssa, add=True)
        gb.wait()
        sb = pltpu.async_copy(bufb, acc_sh.at[col_v.at[b]], ssb, add=True)
        sa.wait()
        sb.wait()
        return _

    lax.fori_loop(0, _CPT // 2, body, None)
    plsc.subcore_barrier()

    for i in range(16):
        pltpu.sync_copy(acc_sh.at[pl.ds(sid * _ROWS_PT + i * 40, 40)], zbuf)
        pltpu.sync_copy(zbuf, out_hbm.at[cid, pl.ds(sid * _ROWS_PT + i * 40, 40)])


@functools.partial(
    pl.kernel,
    mesh=_MESH,
    compiler_params=_SC_PARAMS,
    out_type=(
        jax.ShapeDtypeStruct((2, _NPAD, _H2), jnp.float32),
        jax.ShapeDtypeStruct((2, _NPAD), jnp.float32),
    ),
    scratch_types=[
        pltpu.VMEM((_CPT, _CHUNK), jnp.int32),
        pltpu.VMEM((_CPT, _CHUNK), jnp.int32),
        pltpu.VMEM((_NPAD,), jnp.float32),
        pltpu.VMEM((_NPAD,), jnp.float32),
        pltpu.VMEM((16,), jnp.float32),
        pltpu.VMEM((_CHUNK, _H2), jnp.float32),
        pltpu.VMEM((_CHUNK, _H2), jnp.float32),
        pltpu.VMEM((_CHUNK,), jnp.float32),
        pltpu.VMEM((_CHUNK,), jnp.float32),
        pltpu.VMEM((40, _H2), jnp.float32),
        pltpu.VMEM((_ROWS_PT,), jnp.float32),
        pltpu.VMEM_SHARED((_NPAD, _H2), jnp.float32),
        pltpu.VMEM_SHARED((_NPAD,), jnp.float32),
        pltpu.SemaphoreType.DMA,
        pltpu.SemaphoreType.DMA,
        pltpu.SemaphoreType.DMA,
        pltpu.SemaphoreType.DMA,
        pltpu.SemaphoreType.DMA,
        pltpu.SemaphoreType.DMA,
    ],
)
def _sc_gat(row_hbm, col_hbm, h2_hbm, asrc_hbm, adst_hbm, m_hbm,
            msg_hbm, den_hbm,
            row_v, col_v, asrc_v, adst_v, m_v, bufa, bufb, exa, exb,
            zbuf, zdbuf, accm_sh, accd_sh, sga, sgb, sma, smb, sda, sdb):
    cid = lax.axis_index("c")
    sid = lax.axis_index("s")
    wid = cid * 16 + sid

    pltpu.sync_copy(row_hbm.at[pl.ds(wid * _CPT, _CPT)], row_v)
    pltpu.sync_copy(col_hbm.at[pl.ds(wid * _CPT, _CPT)], col_v)
    pltpu.sync_copy(asrc_hbm, asrc_v)
    pltpu.sync_copy(adst_hbm, adst_v)
    pltpu.sync_copy(m_hbm, m_v)

    z = jnp.zeros((16,), jnp.float32)
    for r in range(40):
        for k in range(_H2 // 16):
            zbuf[r, pl.ds(k * 16, 16)] = z
    _zero16(zdbuf, _ROWS_PT)
    for i in range(16):
        pltpu.sync_copy(zbuf, acc_m_slice(accm_sh, sid, i))
    pltpu.sync_copy(zdbuf, accd_sh.at[pl.ds(sid * _ROWS_PT, _ROWS_PT)])
    plsc.subcore_barrier()

    m = m_v[...]

    def scale(j, buf, ex):
        for k in range(_CHUNK // 16):
            r16 = row_v[j, pl.ds(k * 16, 16)]
            c16 = col_v[j, pl.ds(k * 16, 16)]
            es = plsc.load_gather(asrc_v, [r16])
            ed = plsc.load_gather(adst_v, [c16])
            s = es + ed
            e = jnp.maximum(s, 0.2 * s)
            exv = jnp.exp(e - m)
            ex[pl.ds(k * 16, 16)] = exv
            for l in range(16):
                sp = exv.at[jnp.full((16,), l, jnp.int32)].get(
                    mode="promise_in_bounds")
                q = k * 16 + l
                buf[q, pl.ds(0, 16)] = buf[q, pl.ds(0, 16)] * sp
                buf[q, pl.ds(16, 16)] = buf[q, pl.ds(16, 16)] * sp

    def body(t, _):
        a = 2 * t
        b = a + 1
        ga = pltpu.async_copy(h2_hbm.at[row_v.at[a]], bufa, sga)
        gb = pltpu.async_copy(h2_hbm.at[row_v.at[b]], bufb, sgb)
        ga.wait()
        scale(a, bufa, exa)
        sma_h = pltpu.async_copy(bufa, accm_sh.at[col_v.at[a]], sma, add=True)
        sda_h = pltpu.async_copy(exa, accd_sh.at[col_v.at[a]], sda, add=True)
        gb.wait()
        scale(b, bufb, exb)
        smb_h = pltpu.async_copy(bufb, accm_sh.at[col_v.at[b]], smb, add=True)
        sdb_h = pltpu.async_copy(exb, accd_sh.at[col_v.at[b]], sdb, add=True)
        sma_h.wait()
        sda_h.wait()
        smb_h.wait()
        sdb_h.wait()
        return _

    lax.fori_loop(0, _CPT // 2, body, None)
    plsc.subcore_barrier()

    for i in range(16):
        pltpu.sync_copy(acc_m_slice(accm_sh, sid, i), zbuf)
        pltpu.sync_copy(zbuf, msg_hbm.at[cid, pl.ds(sid * _ROWS_PT + i * 40, 40)])
    pltpu.sync_copy(accd_sh.at[pl.ds(sid * _ROWS_PT, _ROWS_PT)], zdbuf)
    pltpu.sync_copy(zdbuf, den_hbm.at[cid, pl.ds(sid * _ROWS_PT, _ROWS_PT)])


def acc_m_slice(acc, sid, i):
    return acc.at[pl.ds(sid * _ROWS_PT + i * 40, 40)]


def _tc1_body(x_ref, w1_ref, degp_ref, g_ref, h_ref, dinv_ref):
    h = jnp.dot(x_ref[...], w1_ref[...], preferred_element_type=jnp.float32)
    deg = degp_ref[0] + degp_ref[1] + 1.0
    dinv = lax.rsqrt(deg)
    dinv_ref[...] = dinv
    h_ref[...] = h
    g_ref[0:_N, :] = h * dinv[0:_N]
    g_ref[_N:_NPAD, :] = jnp.zeros((_NPAD - _N, _H), jnp.float32)


def _tc2_body(h_ref, dinv_ref, msgp_ref, w2_ref, atts_ref, attd_ref, b1_ref,
              h2_ref, asrc_ref, adst_ref, m_ref):
    h = h_ref[...]
    dinv = dinv_ref[0:_N]
    msg = msgp_ref[0, 0:_N, :] + msgp_ref[1, 0:_N, :]
    h1 = jnp.maximum(dinv * (msg + dinv * h) + b1_ref[...], 0.0)
    h2 = jnp.dot(h1, w2_ref[...], preferred_element_type=jnp.float32)
    a_s = jnp.sum(h2 * atts_ref[...], axis=1, keepdims=True)
    a_d = jnp.sum(h2 * attd_ref[...], axis=1, keepdims=True)
    h2_ref[0:_N, :] = h2
    h2_ref[_N:_NPAD, :] = jnp.zeros((_NPAD - _N, _H2), jnp.float32)
    zcol = jnp.zeros((_NPAD - _N, 1), jnp.float32)
    asrc_ref[0:_N] = a_s
    asrc_ref[_N:_NPAD] = zcol
    adst_ref[0:_N] = a_d
    adst_ref[_N:_NPAD] = zcol
    smax = jnp.max(a_s) + jnp.max(a_d)
    mm = jnp.maximum(smax, 0.2 * smax)
    m_ref[...] = jnp.full((1, 16), mm, jnp.float32)


def _tc3_body(h2_ref, asrc_ref, adst_ref, m_ref, msgp_ref, denp_ref,
              b2_ref, wfc_ref, bfc_ref, out_ref):
    m = m_ref[0, 0]
    s = asrc_ref[0:_N] + adst_ref[0:_N]
    e_self = jnp.maximum(s, 0.2 * s)
    exs = jnp.exp(e_self - m)
    h2 = h2_ref[0:_N, :]
    num = msgp_ref[0, 0:_N, :] + msgp_ref[1, 0:_N, :] + exs * h2
    den = denp_ref[0, 0:_N] + denp_ref[1, 0:_N] + exs + 1e-16
    h3 = jnp.maximum(num / den + b2_ref[...], 0.0)
    out_ref[...] = (
        jnp.dot(h3, wfc_ref[...], preferred_element_type=jnp.float32)
        + bfc_ref[...]
    )


def _tc_call(body, out_shape, *args):
    return pl.pallas_call(body, out_shape=out_shape)(*args)


def kernel(x, edge_index, W1, b1, W2, att_src, att_dst, b2, Wfc, bfc):
    row = edge_index[0].astype(jnp.int32)
    col = edge_index[1].astype(jnp.int32)
    pad = _EPAD - _E
    ar = jnp.arange(pad, dtype=jnp.int32) % 128
    row_p = jnp.concatenate([row, ar]).reshape(_NCHUNKS, _CHUNK)
    col_p = jnp.concatenate([col, _N + ar]).reshape(_NCHUNKS, _CHUNK)

    degp = _sc_degree(col_p)

    g, h, dinv = _tc_call(
        _tc1_body,
        (
            jax.ShapeDtypeStruct((_NPAD, _H), jnp.float32),
            jax.ShapeDtypeStruct((_N, _H), jnp.float32),
            jax.ShapeDtypeStruct((_NPAD, 1), jnp.float32),
        ),
        x, W1, degp.reshape(2, _NPAD, 1),
    )

    msgp = _sc_gcn(row_p, col_p, g)

    h2, asrc, adst, mvec = _tc_call(
        _tc2_body,
        (
            jax.ShapeDtypeStruct((_NPAD, _H2), jnp.float32),
            jax.ShapeDtypeStruct((_NPAD, 1), jnp.float32),
            jax.ShapeDtypeStruct((_NPAD, 1), jnp.float32),
            jax.ShapeDtypeStruct((1, 16), jnp.float32),
        ),
        h, dinv, msgp, W2, att_src.reshape(1, _H2), att_dst.reshape(1, _H2),
        b1.reshape(1, _H),
    )

    msg2p, denp = _sc_gat(
        row_p, col_p, h2, asrc.reshape(_NPAD), adst.reshape(_NPAD),
        mvec.reshape(16),
    )

    out = _tc_call(
        _tc3_body,
        jax.ShapeDtypeStruct((_N, 2), jnp.float32),
        h2, asrc, adst, mvec, msg2p, denp.reshape(2, _NPAD, 1),
        b2.reshape(1, _H2), Wfc, bfc.reshape(1, 2),
    )
    return out


# speedup vs baseline: 62.8644x; 1.0344x over previous
import functools

import jax
import jax.numpy as jnp
from jax import lax
from jax.experimental import pallas as pl
from jax.experimental.pallas import tpu as pltpu
from jax.experimental.pallas import tpu_sc as plsc

_N, _E, _D, _H = 10000, 320000, 128, 64
_H2 = _H // 2

_NPAD = 10240
_CHUNK = 128
_NTILES = 32
_CPT = 80
_EPAD = _NTILES * _CPT * _CHUNK
_NCHUNKS = _EPAD // _CHUNK
_ROWS_PT = _NPAD // 16

_MESH = plsc.VectorSubcoreMesh(core_axis_name="c", subcore_axis_name="s")
_SC_PARAMS = pltpu.CompilerParams(
    needs_layout_passes=False, use_tc_tiling_on_sc=False)


def _zero16(ref, words):
    z = jnp.zeros((16,), jnp.float32)

    def body(i, _):
        ref[pl.ds(i * 16, 16)] = z
        return _

    lax.fori_loop(0, words // 16, body, None)


@functools.partial(
    pl.kernel,
    mesh=_MESH,
    compiler_params=_SC_PARAMS,
    out_type=jax.ShapeDtypeStruct((2, _NPAD), jnp.float32),
    scratch_types=[
        pltpu.VMEM((_CPT, _CHUNK), jnp.int32),
        pltpu.VMEM((_CHUNK,), jnp.float32),
        pltpu.VMEM((_ROWS_PT,), jnp.float32),
        pltpu.VMEM_SHARED((_NPAD,), jnp.float32),
    ],
)
def _sc_degree(col_hbm, out_hbm, col_v, ones_v, buf_v, acc_sh):
    cid = lax.axis_index("c")
    sid = lax.axis_index("s")
    wid = cid * 16 + sid

    pltpu.sync_copy(col_hbm.at[pl.ds(wid * _CPT, _CPT)], col_v)
    one = jnp.ones((16,), jnp.float32)
    for k in range(_CHUNK // 16):
        ones_v[pl.ds(k * 16, 16)] = one
    _zero16(buf_v, _ROWS_PT)
    pltpu.sync_copy(buf_v, acc_sh.at[pl.ds(sid * _ROWS_PT, _ROWS_PT)])
    plsc.subcore_barrier()

    def body(j, _):
        pltpu.sync_copy(ones_v, acc_sh.at[col_v.at[j]], add=True)
        return _

    lax.fori_loop(0, _CPT, body, None)
    plsc.subcore_barrier()

    pltpu.sync_copy(acc_sh.at[pl.ds(sid * _ROWS_PT, _ROWS_PT)], buf_v)
    pltpu.sync_copy(buf_v, out_hbm.at[cid, pl.ds(sid * _ROWS_PT, _ROWS_PT)])


@functools.partial(
    pl.kernel,
    mesh=_MESH,
    compiler_params=_SC_PARAMS,
    out_type=jax.ShapeDtypeStruct((2, _NPAD, _H), jnp.float32),
    scratch_types=[
        pltpu.VMEM((_CPT, _CHUNK), jnp.int32),
        pltpu.VMEM((_CPT, _CHUNK), jnp.int32),
        [pltpu.VMEM((_CHUNK, _H), jnp.float32) for _ in range(4)],
        pltpu.VMEM((40, _H), jnp.float32),
        pltpu.VMEM_SHARED((_NPAD, _H), jnp.float32),
        [pltpu.SemaphoreType.DMA for _ in range(4)],
        [pltpu.SemaphoreType.DMA for _ in range(4)],
    ],
)
def _sc_gcn(row_hbm, col_hbm, g_hbm, out_hbm, row_v, col_v, bufs,
            zbuf, acc_sh, sgs, sss):
    cid = lax.axis_index("c")
    sid = lax.axis_index("s")
    wid = cid * 16 + sid

    pltpu.sync_copy(row_hbm.at[pl.ds(wid * _CPT, _CPT)], row_v)
    pltpu.sync_copy(col_hbm.at[pl.ds(wid * _CPT, _CPT)], col_v)
    z = jnp.zeros((16,), jnp.float32)
    for r in range(40):
        for k in range(_H // 16):
            zbuf[r, pl.ds(k * 16, 16)] = z
    for i in range(16):
        pltpu.sync_copy(zbuf, acc_sh.at[pl.ds(sid * _ROWS_PT + i * 40, 40)])
    plsc.subcore_barrier()

    def body(t, _):
        base = 4 * t
        gh = [
            pltpu.async_copy(g_hbm.at[row_v.at[base + i]], bufs[i], sgs[i])
            for i in range(4)
        ]
        sh = []
        for i in range(4):
            gh[i].wait()
            sh.append(pltpu.async_copy(
                bufs[i], acc_sh.at[col_v.at[base + i]], sss[i], add=True))
        for h in sh:
            h.wait()
        return _

    lax.fori_loop(0, _CPT // 4, body, None)
    plsc.subcore_barrier()

    for i in range(16):
        pltpu.sync_copy(acc_sh.at[pl.ds(sid * _ROWS_PT + i * 40, 40)], zbuf)
        pltpu.sync_copy(zbuf, out_hbm.at[cid, pl.ds(sid * _ROWS_PT + i * 40, 40)])


@functools.partial(
    pl.kernel,
    mesh=_MESH,
    compiler_params=_SC_PARAMS,
    out_type=(
        jax.ShapeDtypeStruct((2, _NPAD, _H2), jnp.float32),
        jax.ShapeDtypeStruct((2, _NPAD), jnp.float32),
    ),
    scratch_types=[
        pltpu.VMEM((_CPT, _CHUNK), jnp.int32),
        pltpu.VMEM((_CPT, _CHUNK), jnp.int32),
        pltpu.VMEM((_NPAD,), jnp.float32),
        pltpu.VMEM((_NPAD,), jnp.float32),
        pltpu.VMEM((16,), jnp.float32),
        pltpu.VMEM((_CHUNK, _H2), jnp.float32),
        pltpu.VMEM((_CHUNK, _H2), jnp.float32),
        pltpu.VMEM((_CHUNK,), jnp.float32),
        pltpu.VMEM((_CHUNK,), jnp.float32),
        pltpu.VMEM((40, _H2), jnp.float32),
        pltpu.VMEM((_ROWS_PT,), jnp.float32),
        pltpu.VMEM_SHARED((_NPAD, _H2), jnp.float32),
        pltpu.VMEM_SHARED((_NPAD,), jnp.float32),
        pltpu.SemaphoreType.DMA,
        pltpu.SemaphoreType.DMA,
        pltpu.SemaphoreType.DMA,
        pltpu.SemaphoreType.DMA,
        pltpu.SemaphoreType.DMA,
        pltpu.SemaphoreType.DMA,
    ],
)
def _sc_gat(row_hbm, col_hbm, h2_hbm, asrc_hbm, adst_hbm, m_hbm,
            msg_hbm, den_hbm,
            row_v, col_v, asrc_v, adst_v, m_v, bufa, bufb, exa, exb,
            zbuf, zdbuf, accm_sh, accd_sh, sga, sgb, sma, smb, sda, sdb):
    cid = lax.axis_index("c")
    sid = lax.axis_index("s")
    wid = cid * 16 + sid

    pltpu.sync_copy(row_hbm.at[pl.ds(wid * _CPT, _CPT)], row_v)
    pltpu.sync_copy(col_hbm.at[pl.ds(wid * _CPT, _CPT)], col_v)
    pltpu.sync_copy(asrc_hbm, asrc_v)
    pltpu.sync_copy(adst_hbm, adst_v)
    pltpu.sync_copy(m_hbm, m_v)

    z = jnp.zeros((16,), jnp.float32)
    for r in range(40):
        for k in range(_H2 // 16):
            zbuf[r, pl.ds(k * 16, 16)] = z
    _zero16(zdbuf, _ROWS_PT)
    for i in range(16):
        pltpu.sync_copy(zbuf, acc_m_slice(accm_sh, sid, i))
    pltpu.sync_copy(zdbuf, accd_sh.at[pl.ds(sid * _ROWS_PT, _ROWS_PT)])
    plsc.subcore_barrier()

    m = m_v[...]

    def scale(j, buf, ex):
        for k in range(_CHUNK // 16):
            r16 = row_v[j, pl.ds(k * 16, 16)]
            c16 = col_v[j, pl.ds(k * 16, 16)]
            es = plsc.load_gather(asrc_v, [r16])
            ed = plsc.load_gather(adst_v, [c16])
            s = es + ed
            e = jnp.maximum(s, 0.2 * s)
            exv = jnp.exp(e - m)
            ex[pl.ds(k * 16, 16)] = exv
            for l in range(16):
                sp = exv.at[jnp.full((16,), l, jnp.int32)].get(
                    mode="promise_in_bounds")
                q = k * 16 + l
                buf[q, pl.ds(0, 16)] = buf[q, pl.ds(0, 16)] * sp
                buf[q, pl.ds(16, 16)] = buf[q, pl.ds(16, 16)] * sp

    def body(t, _):
        a = 2 * t
        b = a + 1
        ga = pltpu.async_copy(h2_hbm.at[row_v.at[a]], bufa, sga)
        gb = pltpu.async_copy(h2_hbm.at[row_v.at[b]], bufb, sgb)
        ga.wait()
        scale(a, bufa, exa)
        sma_h = pltpu.async_copy(bufa, accm_sh.at[col_v.at[a]], sma, add=True)
        sda_h = pltpu.async_copy(exa, accd_sh.at[col_v.at[a]], sda, add=True)
        gb.wait()
        scale(b, bufb, exb)
        smb_h = pltpu.async_copy(bufb, accm_sh.at[col_v.at[b]], smb, add=True)
        sdb_h = pltpu.async_copy(exb, accd_sh.at[col_v.at[b]], sdb, add=True)
        sma_h.wait()
        sda_h.wait()
        smb_h.wait()
        sdb_h.wait()
        return _

    lax.fori_loop(0, _CPT // 2, body, None)
    plsc.subcore_barrier()

    for i in range(16):
        pltpu.sync_copy(acc_m_slice(accm_sh, sid, i), zbuf)
        pltpu.sync_copy(zbuf, msg_hbm.at[cid, pl.ds(sid * _ROWS_PT + i * 40, 40)])
    pltpu.sync_copy(accd_sh.at[pl.ds(sid * _ROWS_PT, _ROWS_PT)], zdbuf)
    pltpu.sync_copy(zdbuf, den_hbm.at[cid, pl.ds(sid * _ROWS_PT, _ROWS_PT)])


def acc_m_slice(acc, sid, i):
    return acc.at[pl.ds(sid * _ROWS_PT + i * 40, 40)]


def _tc_mm_body(x_ref, w1_ref, h_ref):
    h_ref[...] = jnp.dot(
        x_ref[...], w1_ref[...], preferred_element_type=jnp.float32)


def _tc1_body(h_ref, degp_ref, g_ref, dinv_ref):
    h = h_ref[...]
    deg = degp_ref[0] + degp_ref[1] + 1.0
    dinv = lax.rsqrt(deg)
    dinv_ref[...] = dinv
    g_ref[0:_N, :] = h * dinv[0:_N]
    g_ref[_N:_NPAD, :] = jnp.zeros((_NPAD - _N, _H), jnp.float32)


def _tc2_body(h_ref, dinv_ref, msgp_ref, w2_ref, atts_ref, attd_ref, b1_ref,
              h2_ref, asrc_ref, adst_ref, m_ref):
    h = h_ref[...]
    dinv = dinv_ref[0:_N]
    msg = msgp_ref[0, 0:_N, :] + msgp_ref[1, 0:_N, :]
    h1 = jnp.maximum(dinv * (msg + dinv * h) + b1_ref[...], 0.0)
    h2 = jnp.dot(h1, w2_ref[...], preferred_element_type=jnp.float32)
    a_s = jnp.sum(h2 * atts_ref[...], axis=1, keepdims=True)
    a_d = jnp.sum(h2 * attd_ref[...], axis=1, keepdims=True)
    h2_ref[0:_N, :] = h2
    h2_ref[_N:_NPAD, :] = jnp.zeros((_NPAD - _N, _H2), jnp.float32)
    zcol = jnp.zeros((_NPAD - _N, 1), jnp.float32)
    asrc_ref[0:_N] = a_s
    asrc_ref[_N:_NPAD] = zcol
    adst_ref[0:_N] = a_d
    adst_ref[_N:_NPAD] = zcol
    smax = jnp.max(a_s) + jnp.max(a_d)
    mm = jnp.maximum(smax, 0.2 * smax)
    m_ref[...] = jnp.full((1, 16), mm, jnp.float32)


def _tc3_body(h2_ref, asrc_ref, adst_ref, m_ref, msgp_ref, denp_ref,
              b2_ref, wfc_ref, bfc_ref, out_ref):
    m = m_ref[0, 0]
    s = asrc_ref[0:_N] + adst_ref[0:_N]
    e_self = jnp.maximum(s, 0.2 * s)
    exs = jnp.exp(e_self - m)
    h2 = h2_ref[0:_N, :]
    num = msgp_ref[0, 0:_N, :] + msgp_ref[1, 0:_N, :] + exs * h2
    den = denp_ref[0, 0:_N] + denp_ref[1, 0:_N] + exs + 1e-16
    h3 = jnp.maximum(num / den + b2_ref[...], 0.0)
    out_ref[...] = (
        jnp.dot(h3, wfc_ref[...], preferred_element_type=jnp.float32)
        + bfc_ref[...]
    )


def _tc_call(body, out_shape, *args):
    return pl.pallas_call(body, out_shape=out_shape)(*args)


def kernel(x, edge_index, W1, b1, W2, att_src, att_dst, b2, Wfc, bfc):
    row = edge_index[0].astype(jnp.int32)
    col = edge_index[1].astype(jnp.int32)
    pad = _EPAD - _E
    ar = jnp.arange(pad, dtype=jnp.int32) % 128
    row_p = jnp.concatenate([row, ar]).reshape(_NCHUNKS, _CHUNK)
    col_p = jnp.concatenate([col, _N + ar]).reshape(_NCHUNKS, _CHUNK)

    degp = _sc_degree(col_p)

    h = _tc_call(
        _tc_mm_body,
        jax.ShapeDtypeStruct((_N, _H), jnp.float32),
        x, W1,
    )

    g, dinv = _tc_call(
        _tc1_body,
        (
            jax.ShapeDtypeStruct((_NPAD, _H), jnp.float32),
            jax.ShapeDtypeStruct((_NPAD, 1), jnp.float32),
        ),
        h, degp.reshape(2, _NPAD, 1),
    )

    msgp = _sc_gcn(row_p, col_p, g)

    h2, asrc, adst, mvec = _tc_call(
        _tc2_body,
        (
            jax.ShapeDtypeStruct((_NPAD, _H2), jnp.float32),
            jax.ShapeDtypeStruct((_NPAD, 1), jnp.float32),
            jax.ShapeDtypeStruct((_NPAD, 1), jnp.float32),
            jax.ShapeDtypeStruct((1, 16), jnp.float32),
        ),
        h, dinv, msgp, W2, att_src.reshape(1, _H2), att_dst.reshape(1, _H2),
        b1.reshape(1, _H),
    )

    msg2p, denp = _sc_gat(
        row_p, col_p, h2, asrc.reshape(_NPAD), adst.reshape(_NPAD),
        mvec.reshape(16),
    )

    out = _tc_call(
        _tc3_body,
        jax.ShapeDtypeStruct((_N, 2), jnp.float32),
        h2, asrc, adst, mvec, msg2p, denp.reshape(2, _NPAD, 1),
        b2.reshape(1, _H2), Wfc, bfc.reshape(1, 2),
    )
    return out
